# baseline (speedup 1.0000x reference)
import functools

import jax
import jax.numpy as jnp
from jax import lax
from jax.experimental import pallas as pl
from jax.experimental.pallas import tpu as pltpu

N = 50000
E = 800000
HIDDEN = 64
EMBED = 16
STATE = 4
SIZE = 2
CLASS_NUM = 10
COND = 2 * EMBED

NW = 32
NPT = 1568
N2 = NW * NPT
EP = 802816
NEG = -3.0e38


def _node0_body(x_ref, geo_ref, t_ref, cat_ref, wf_ref, sigw_ref, sigb_ref,
                emb_ref, e1w_ref, e1b_ref, e2w_ref, e2b_ref,
                w1a_ref, w1b_ref, b1_ref,
                a_ref, b_out_ref, cond_ref, inv_ref):
    ts = t_ref[...]
    xp = ts * wf_ref[...] * (2.0 * jnp.pi)
    four = jnp.concatenate([jnp.sin(xp), jnp.cos(xp)], axis=-1)
    sig = jnp.maximum(jnp.dot(four, sigw_ref[...]) + sigb_ref[...], 0.0)
    cat = cat_ref[...]
    onehot = (cat == lax.broadcasted_iota(jnp.int32, (1, CLASS_NUM), 1)
              ).astype(jnp.float32)
    cls = jnp.maximum(jnp.dot(onehot, emb_ref[...]), 0.0)
    cond = jnp.concatenate([cls, sig], axis=-1)
    h = jnp.concatenate([x_ref[...], geo_ref[...]], axis=-1)
    h = jnp.maximum(jnp.dot(h, e1w_ref[...]) + e1b_ref[...], 0.0)
    h = jnp.maximum(jnp.dot(h, e2w_ref[...]) + e2b_ref[...], 0.0)
    x1 = jnp.concatenate([h, cond], axis=-1)
    a_ref[...] = jnp.dot(x1, w1a_ref[...]) + b1_ref[...]
    b_out_ref[...] = jnp.dot(x1, w1b_ref[...])
    cond_ref[...] = cond
    l25 = jnp.log(25.0)
    std = jnp.sqrt((jnp.exp(2.0 * l25 * ts) - 1.0) / (2.0 * l25))
    inv_ref[...] = 1.0 / (std + 1e-7)


def _tc_node0(xp, geop, tp, catp, wf, sigW, sigb, emb, e1W, e1b, e2W, e2b,
              W1A, W1B, b1):
    BN = 6272
    grid = (N2 // BN,)
    row = lambda i: (i, 0)
    full = lambda i: (0, 0)
    return pl.pallas_call(
        _node0_body,
        grid=grid,
        in_specs=[
            pl.BlockSpec((BN, STATE), row),
            pl.BlockSpec((BN, SIZE), row),
            pl.BlockSpec((BN, 1), row),
            pl.BlockSpec((BN, 1), row),
            pl.BlockSpec((1, EMBED // 2), full),
            pl.BlockSpec((EMBED, EMBED), full),
            pl.BlockSpec((1, EMBED), full),
            pl.BlockSpec((CLASS_NUM, EMBED), full),
            pl.BlockSpec((STATE + SIZE, HIDDEN), full),
            pl.BlockSpec((1, HIDDEN), full),
            pl.BlockSpec((HIDDEN, HIDDEN), full),
            pl.BlockSpec((1, HIDDEN), full),
            pl.BlockSpec((HIDDEN + COND, HIDDEN), full),
            pl.BlockSpec((HIDDEN + COND, HIDDEN), full),
            pl.BlockSpec((1, HIDDEN), full),
        ],
        out_specs=[
            pl.BlockSpec((BN, HIDDEN), row),
            pl.BlockSpec((BN, HIDDEN), row),
            pl.BlockSpec((BN, COND), row),
            pl.BlockSpec((BN, 1), row),
        ],
        out_shape=[
            jax.ShapeDtypeStruct((N2, HIDDEN), jnp.float32),
            jax.ShapeDtypeStruct((N2, HIDDEN), jnp.float32),
            jax.ShapeDtypeStruct((N2, COND), jnp.float32),
            jax.ShapeDtypeStruct((N2, 1), jnp.float32),
        ],
    )(xp, geop, tp, catp, wf, sigW, sigb, emb, e1W, e1b, e2W, e2b, W1A, W1B, b1)


def _node2_body(o1_ref, cond_ref, w2a_ref, w2b_ref, b2_ref, a_ref, b_out_ref):
    x2 = jnp.concatenate([jnp.maximum(o1_ref[...], 0.0), cond_ref[...]],
                         axis=-1)
    a_ref[...] = jnp.dot(x2, w2a_ref[...]) + b2_ref[...]
    b_out_ref[...] = jnp.dot(x2, w2b_ref[...])


def _tc_node2(out1, cond, W2A, W2B, b2):
    BN = 6272
    grid = (N2 // BN,)
    row = lambda i: (i, 0)
    full = lambda i: (0, 0)
    return pl.pallas_call(
        _node2_body,
        grid=grid,
        in_specs=[
            pl.BlockSpec((BN, HIDDEN), row),
            pl.BlockSpec((BN, COND), row),
            pl.BlockSpec((HIDDEN + COND, HIDDEN), full),
            pl.BlockSpec((HIDDEN + COND, HIDDEN), full),
            pl.BlockSpec((1, HIDDEN), full),
        ],
        out_specs=[
            pl.BlockSpec((BN, HIDDEN), row),
            pl.BlockSpec((BN, HIDDEN), row),
        ],
        out_shape=[
            jax.ShapeDtypeStruct((N2, HIDDEN), jnp.float32),
            jax.ShapeDtypeStruct((N2, HIDDEN), jnp.float32),
        ],
    )(out1, cond, W2A, W2B, b2)


def _edge_body(pa_ref, pb_ref, w_ref, b_ref, m_ref):
    g = jnp.maximum(pa_ref[...] + pb_ref[...], 0.0)
    m_ref[...] = jnp.dot(g, w_ref[...]) + b_ref[...]


def _tc_edge(PA, PB, W2, b2, dout):
    BE = 2048
    grid = (EP // BE,)
    row = lambda i: (i, 0)
    full = lambda i: (0, 0)
    return pl.pallas_call(
        _edge_body,
        grid=grid,
        in_specs=[
            pl.BlockSpec((BE, HIDDEN), row),
            pl.BlockSpec((BE, HIDDEN), row),
            pl.BlockSpec((HIDDEN, dout), full),
            pl.BlockSpec((1, dout), full),
        ],
        out_specs=pl.BlockSpec((BE, dout), row),
        out_shape=jax.ShapeDtypeStruct((EP, dout), jnp.float32),
    )(PA, PB, W2, b2)


def kernel(x, geo, t, category, edge_index, W_fourier, sigW, sigb, emb_table,
           enc1W, enc1b, enc2W, enc2b, m1aW, m1ab, m1bW, m1bb,
           m2aW, m2ab, m2bW, m2bb):
    f32 = jnp.float32
    pad_n = lambda a: jnp.pad(a, ((0, N2 - N), (0, 0)))
    xp = pad_n(x)
    geop = pad_n(geo)
    tp = jnp.pad(t, ((0, N2 - N), (0, 0)), constant_values=0.5)
    catp = jnp.pad(category[:, None], ((0, N2 - N), (0, 0)))
    wf = W_fourier[None, :]
    sigb2 = sigb[None, :]
    e1b2 = enc1b[None, :]
    e2b2 = enc2b[None, :]
    W1A = m1aW[:HIDDEN + COND] - m1aW[HIDDEN + COND:]
    W1B = m1aW[HIDDEN + COND:]
    W2A = m2aW[:HIDDEN + COND] - m2aW[HIDDEN + COND:]
    W2B = m2aW[HIDDEN + COND:]

    A1, B1, cond, invstd = _tc_node0(
        xp, geop, tp, catp, wf, sigW, sigb2, emb_table,
        enc1W, e1b2, enc2W, e2b2, W1A, W1B, m1ab[None, :])

    src = edge_index[0]
    dst = edge_index[1]

    PA1 = jnp.take(A1, dst, axis=0)
    PB1 = jnp.take(B1, src, axis=0)
    PA1 = jnp.pad(PA1, ((0, EP - E), (0, 0)))
    PB1 = jnp.pad(PB1, ((0, EP - E), (0, 0)))
    M1 = _tc_edge(PA1, PB1, m1bW, m1bb[None, :], HIDDEN)
    o1 = jax.ops.segment_max(M1[:E], dst, num_segments=N2)
    o1 = jnp.where(jnp.isfinite(o1), o1, 0.0)

    A2, B2 = _tc_node2(o1, cond, W2A, W2B, m2ab[None, :])

    PA2 = jnp.take(A2, dst, axis=0)
    PB2 = jnp.take(B2, src, axis=0)
    PA2 = jnp.pad(PA2, ((0, EP - E), (0, 0)))
    PB2 = jnp.pad(PB2, ((0, EP - E), (0, 0)))
    M2 = _tc_edge(PA2, PB2, m2bW, m2bb[None, :], STATE)
    o2 = jax.ops.segment_max(M2[:E], dst, num_segments=N2)
    o2 = jnp.where(jnp.isfinite(o2), o2, 0.0)
    out = o2 * invstd
    return out[:N].astype(f32)


# speedup vs baseline: 1.9250x; 1.9250x over previous
import functools

import jax
import jax.numpy as jnp
from jax import lax
from jax.experimental import pallas as pl
from jax.experimental.pallas import tpu as pltpu
from jax.experimental.pallas import tpu_sc as plsc

N = 50000
E = 800000
HIDDEN = 64
EMBED = 16
STATE = 4
SIZE = 2
CLASS_NUM = 10
COND = 2 * EMBED

NW = 32
NPT = 1568
N2 = NW * NPT
EP = 802816
NEG = -3.0e38


def _node0_body(x_ref, geo_ref, t_ref, cat_ref, wf_ref, sigw_ref, sigb_ref,
                emb_ref, e1w_ref, e1b_ref, e2w_ref, e2b_ref,
                w1a_ref, w1b_ref, b1_ref,
                a_ref, b_out_ref, cond_ref, inv_ref):
    ts = t_ref[...]
    xp = ts * wf_ref[...] * (2.0 * jnp.pi)
    four = jnp.concatenate([jnp.sin(xp), jnp.cos(xp)], axis=-1)
    sig = jnp.maximum(jnp.dot(four, sigw_ref[...]) + sigb_ref[...], 0.0)
    cat = cat_ref[...]
    onehot = (cat == lax.broadcasted_iota(jnp.int32, (1, CLASS_NUM), 1)
              ).astype(jnp.float32)
    cls = jnp.maximum(jnp.dot(onehot, emb_ref[...]), 0.0)
    cond = jnp.concatenate([cls, sig], axis=-1)
    h = jnp.concatenate([x_ref[...], geo_ref[...]], axis=-1)
    h = jnp.maximum(jnp.dot(h, e1w_ref[...]) + e1b_ref[...], 0.0)
    h = jnp.maximum(jnp.dot(h, e2w_ref[...]) + e2b_ref[...], 0.0)
    x1 = jnp.concatenate([h, cond], axis=-1)
    a_ref[...] = jnp.dot(x1, w1a_ref[...]) + b1_ref[...]
    b_out_ref[...] = jnp.dot(x1, w1b_ref[...])
    cond_ref[...] = cond
    l25 = jnp.log(25.0)
    std = jnp.sqrt((jnp.exp(2.0 * l25 * ts) - 1.0) / (2.0 * l25))
    inv_ref[...] = 1.0 / (std + 1e-7)


def _tc_node0(xp, geop, tp, catp, wf, sigW, sigb, emb, e1W, e1b, e2W, e2b,
              W1A, W1B, b1):
    BN = 6272
    grid = (N2 // BN,)
    row = lambda i: (i, 0)
    full = lambda i: (0, 0)
    return pl.pallas_call(
        _node0_body,
        grid=grid,
        in_specs=[
            pl.BlockSpec((BN, STATE), row),
            pl.BlockSpec((BN, SIZE), row),
            pl.BlockSpec((BN, 1), row),
            pl.BlockSpec((BN, 1), row),
            pl.BlockSpec((1, EMBED // 2), full),
            pl.BlockSpec((EMBED, EMBED), full),
            pl.BlockSpec((1, EMBED), full),
            pl.BlockSpec((CLASS_NUM, EMBED), full),
            pl.BlockSpec((STATE + SIZE, HIDDEN), full),
            pl.BlockSpec((1, HIDDEN), full),
            pl.BlockSpec((HIDDEN, HIDDEN), full),
            pl.BlockSpec((1, HIDDEN), full),
            pl.BlockSpec((HIDDEN + COND, HIDDEN), full),
            pl.BlockSpec((HIDDEN + COND, HIDDEN), full),
            pl.BlockSpec((1, HIDDEN), full),
        ],
        out_specs=[
            pl.BlockSpec((BN, HIDDEN), row),
            pl.BlockSpec((BN, HIDDEN), row),
            pl.BlockSpec((BN, COND), row),
            pl.BlockSpec((BN, 1), row),
        ],
        out_shape=[
            jax.ShapeDtypeStruct((N2, HIDDEN), jnp.float32),
            jax.ShapeDtypeStruct((N2, HIDDEN), jnp.float32),
            jax.ShapeDtypeStruct((N2, COND), jnp.float32),
            jax.ShapeDtypeStruct((N2, 1), jnp.float32),
        ],
    )(xp, geop, tp, catp, wf, sigW, sigb, emb, e1W, e1b, e2W, e2b, W1A, W1B, b1)


def _node2_body(o1_ref, cond_ref, w2a_ref, w2b_ref, b2_ref, a_ref, b_out_ref):
    x2 = jnp.concatenate([jnp.maximum(o1_ref[...], 0.0), cond_ref[...]],
                         axis=-1)
    a_ref[...] = jnp.dot(x2, w2a_ref[...]) + b2_ref[...]
    b_out_ref[...] = jnp.dot(x2, w2b_ref[...])


def _tc_node2(out1, cond, W2A, W2B, b2):
    BN = 6272
    grid = (N2 // BN,)
    row = lambda i: (i, 0)
    full = lambda i: (0, 0)
    return pl.pallas_call(
        _node2_body,
        grid=grid,
        in_specs=[
            pl.BlockSpec((BN, HIDDEN), row),
            pl.BlockSpec((BN, COND), row),
            pl.BlockSpec((HIDDEN + COND, HIDDEN), full),
            pl.BlockSpec((HIDDEN + COND, HIDDEN), full),
            pl.BlockSpec((1, HIDDEN), full),
        ],
        out_specs=[
            pl.BlockSpec((BN, HIDDEN), row),
            pl.BlockSpec((BN, HIDDEN), row),
        ],
        out_shape=[
            jax.ShapeDtypeStruct((N2, HIDDEN), jnp.float32),
            jax.ShapeDtypeStruct((N2, HIDDEN), jnp.float32),
        ],
    )(out1, cond, W2A, W2B, b2)


def _edge_body(pa_ref, pb_ref, w_ref, b_ref, m_ref):
    g = jnp.maximum(pa_ref[...] + pb_ref[...], 0.0)
    m_ref[...] = jnp.dot(g, w_ref[...]) + b_ref[...]


def _tc_edge(PA, PB, W2, b2, dout):
    BE = 2048
    grid = (EP // BE,)
    row = lambda i: (i, 0)
    full = lambda i: (0, 0)
    return pl.pallas_call(
        _edge_body,
        grid=grid,
        in_specs=[
            pl.BlockSpec((BE, HIDDEN), row),
            pl.BlockSpec((BE, HIDDEN), row),
            pl.BlockSpec((HIDDEN, dout), full),
            pl.BlockSpec((1, dout), full),
        ],
        out_specs=pl.BlockSpec((BE, dout), row),
        out_shape=jax.ShapeDtypeStruct((EP, dout), jnp.float32),
    )(PA, PB, W2, b2)


_MESH = functools.partial(plsc.VectorSubcoreMesh, core_axis_name="c",
                          subcore_axis_name="s", num_cores=2, num_subcores=16)

EW = E // NW
GW = 200


def _sc_gather_body(a_hbm, b_hbm, d_hbm, s_hbm, pa_hbm, pb_hbm,
                    idxd, idxs, rowsa, rowsb, sema, semb):
    wid = lax.axis_index("s") * 2 + lax.axis_index("c")
    base = wid * EW

    def step(g, carry):
        start = base + g * GW
        pltpu.sync_copy(d_hbm.at[pl.ds(start, GW)], idxd)
        pltpu.sync_copy(s_hbm.at[pl.ds(start, GW)], idxs)
        cpa = pltpu.async_copy(a_hbm.at[idxd], rowsa, sema)
        cpb = pltpu.async_copy(b_hbm.at[idxs], rowsb, semb)
        cpa.wait()
        cpb.wait()
        pltpu.sync_copy(rowsa, pa_hbm.at[pl.ds(start, GW), :])
        pltpu.sync_copy(rowsb, pb_hbm.at[pl.ds(start, GW), :])
        return carry

    lax.fori_loop(0, EW // GW, step, 0)


def _sc_gather(A, B, dR, sR):
    k = pl.kernel(
        _sc_gather_body,
        out_type=[
            jax.ShapeDtypeStruct((EP, HIDDEN), jnp.float32),
            jax.ShapeDtypeStruct((EP, HIDDEN), jnp.float32),
        ],
        mesh=_MESH(),
        scratch_types=[
            pltpu.VMEM((GW,), jnp.int32),
            pltpu.VMEM((GW,), jnp.int32),
            pltpu.VMEM((GW, HIDDEN), jnp.float32),
            pltpu.VMEM((GW, HIDDEN), jnp.float32),
            pltpu.SemaphoreType.DMA,
            pltpu.SemaphoreType.DMA,
        ],
        compiler_params=pltpu.CompilerParams(use_tc_tiling_on_sc=False, needs_layout_passes=False),
    )
    return k(A, B, dR, sR)


HW = 1024
NHG = HW // 16
NWIN = (EW + HW - 1) // HW
EPAD_IN = E + HW * NW // 32 + 1024


def _bucket(d):
    return ((d >> 5) * 2675) >> 17


def _iota():
    return lax.iota(jnp.int32, 16)


def _sc_hist_body(d_hbm, cnt_hbm, loc, win, sem):
    wid = lax.axis_index("s") * 2 + lax.axis_index("c")
    base = wid * EW
    zero = jnp.zeros((16,), jnp.int32)
    for b in range(32):
        loc[pl.ds(b * 16, 16)] = zero

    def step(g, carry):
        pltpu.sync_copy(d_hbm.at[pl.ds(base + g * HW, HW)], win)
        rem = EW - g * HW

        def group(i, c2):
            msk = (i * 16 + _iota()) < rem
            d = win[pl.ds(i * 16, 16)]
            addr = _bucket(d) * 16 + _iota()
            cur = plsc.load_gather(loc, [addr], mask=msk)
            plsc.store_scatter(loc, [addr], cur + 1, mask=msk)
            return c2

        lax.fori_loop(0, NHG, group, 0)
        return carry

    lax.fori_loop(0, NWIN, step, 0)
    for b in range(32):
        pltpu.sync_copy(loc.at[pl.ds(b * 16, 16)],
                        cnt_hbm.at[pl.ds(b * NW * 16 + wid * 16, 16)])


def _sc_hist(dst_p):
    k = pl.kernel(
        _sc_hist_body,
        out_type=jax.ShapeDtypeStruct((32 * NW * 16,), jnp.int32),
        mesh=_MESH(),
        scratch_types=[
            pltpu.VMEM((32 * 16,), jnp.int32),
            pltpu.VMEM((HW,), jnp.int32),
            pltpu.SemaphoreType.DMA,
        ],
        compiler_params=pltpu.CompilerParams(use_tc_tiling_on_sc=False, needs_layout_passes=False),
    )
    return k(dst_p)


def _sc_route_body(d_hbm, s_hbm, cnt_hbm, dr_hbm, sr_hbm, offs_hbm,
                   scan, dwin, swin, pos, offsv, sem, semd, sems):
    wid = lax.axis_index("s") * 2 + lax.axis_index("c")
    base = wid * EW
    pltpu.sync_copy(cnt_hbm, scan)

    def scan_step(k, carry):
        v = plsc.load_gather(scan, [k * 16 + _iota()])
        s = plsc.cumsum(v)
        excl = s - v + carry
        plsc.store_scatter(scan, [k * 16 + _iota()], excl)
        return carry + lax.reduce_max(s, (0,))

    lax.fori_loop(0, 32 * NW, scan_step, jnp.int32(0))

    o0 = plsc.load_gather(scan, [_iota() * (NW * 16)])
    o1 = plsc.load_gather(scan, [(16 + _iota()) * (NW * 16)])
    offsv[pl.ds(0, 16)] = o0
    offsv[pl.ds(16, 16)] = o1
    offsv[pl.ds(32, 16)] = jnp.full((16,), E, jnp.int32)

    @pl.when(wid == 0)
    def _():
        pltpu.sync_copy(offsv, offs_hbm)

    trash = E + wid * 16 + _iota()

    def step(g, carry):
        gbase = base + g * HW
        pltpu.sync_copy(d_hbm.at[pl.ds(gbase, HW)], dwin)
        pltpu.sync_copy(s_hbm.at[pl.ds(gbase, HW)], swin)
        rem = EW - g * HW

        def group(i, c2):
            msk = (i * 16 + _iota()) < rem
            d = dwin[pl.ds(i * 16, 16)]
            addr = _bucket(d) * (NW * 16) + wid * 16 + _iota()
            p = plsc.load_gather(scan, [addr], mask=msk)
            plsc.store_scatter(scan, [addr], p + 1, mask=msk)
            plsc.store_scatter(pos, [i * 16 + _iota()], jnp.where(msk, p, trash))
            return c2

        lax.fori_loop(0, NHG, group, 0)
        cpd = pltpu.async_copy(dwin, dr_hbm.at[pos], semd)
        cps = pltpu.async_copy(swin, sr_hbm.at[pos], sems)
        cpd.wait()
        cps.wait()
        return carry

    lax.fori_loop(0, NWIN, step, 0)


def _sc_route(dst_p, src_p, counts):
    k = pl.kernel(
        _sc_route_body,
        out_type=[
            jax.ShapeDtypeStruct((EP,), jnp.int32),
            jax.ShapeDtypeStruct((EP,), jnp.int32),
            jax.ShapeDtypeStruct((48,), jnp.int32),
        ],
        mesh=_MESH(),
        scratch_types=[
            pltpu.VMEM((32 * NW * 16,), jnp.int32),
            pltpu.VMEM((HW,), jnp.int32),
            pltpu.VMEM((HW,), jnp.int32),
            pltpu.VMEM((HW,), jnp.int32),
            pltpu.VMEM((48,), jnp.int32),
            pltpu.SemaphoreType.DMA,
            pltpu.SemaphoreType.DMA,
            pltpu.SemaphoreType.DMA,
        ],
        compiler_params=pltpu.CompilerParams(use_tc_tiling_on_sc=False, needs_layout_passes=False),
    )
    return k(dst_p, src_p, counts)


WS = 256
NSG = WS // 16


def _scatter_body(m_hbm, dr_hbm, offs_hbm, inv_hbm, out_hbm,
                  acc, dwin, mwin, offsv, invv, sem, d):
    wid = lax.axis_index("s") * 2 + lax.axis_index("c")
    nbase = wid * NPT
    pltpu.sync_copy(offs_hbm, offsv)
    pltpu.sync_copy(inv_hbm.at[pl.ds(nbase, NPT)], invv)
    widv = jnp.full((16,), wid, jnp.int32)
    off_s = lax.reduce_max(plsc.load_gather(offsv, [widv]), (0,))
    end_s = lax.reduce_max(plsc.load_gather(offsv, [widv + 1]), (0,))
    start0 = (off_s // 8) * 8
    nwin = (end_s - start0 + WS - 1) // WS

    ninf = jnp.full((16,), -jnp.inf, jnp.float32)

    def initstep(k, carry):
        plsc.store_scatter(acc, [k * 16 + _iota()], ninf)
        return carry

    lax.fori_loop(0, NPT * d // 16, initstep, 0)

    def win(g, carry):
        wbase = start0 + g * WS
        pltpu.sync_copy(dr_hbm.at[pl.ds(wbase, WS)], dwin)
        pltpu.sync_copy(m_hbm.at[pl.ds(wbase * d, WS * d)], mwin)

        def group(i, c2):
            dloc = dwin[pl.ds(i * 16, 16)] - nbase
            ebase = wbase + i * 16
            for j in range(16):
                ok = jnp.logical_and(ebase + j >= off_s, ebase + j < end_s)
                mskj = jnp.full((16,), False) | ok
                dj = lax.reduce_max(
                    jnp.where(_iota() == j, dloc, -(2 ** 30)), (0,))
                rowa = dj * d
                rowm = (i * 16 + j) * d
                if d >= 16:
                    for c in range(d // 16):
                        addr = rowa + c * 16 + _iota()
                        cur = plsc.load_gather(acc, [addr], mask=mskj)
                        mv = mwin[pl.ds(rowm + c * 16, 16)]
                        plsc.store_scatter(acc, [addr],
                                           jnp.maximum(cur, mv), mask=mskj)
                else:
                    mk = jnp.logical_and(mskj, _iota() < d)
                    addr = rowa + _iota()
                    cur = plsc.load_gather(acc, [addr], mask=mk)
                    mv = plsc.load_gather(mwin, [rowm + _iota()], mask=mk)
                    plsc.store_scatter(acc, [addr],
                                       jnp.maximum(cur, mv), mask=mk)
            return c2

        lax.fori_loop(0, NSG, group, 0)
        return carry

    lax.fori_loop(0, nwin, win, 0)

    def finstep(k, carry):
        idx = k * 16 + _iota()
        v = plsc.load_gather(acc, [idx])
        iv = plsc.load_gather(invv, [idx // d])
        v = jnp.where(v == ninf, jnp.zeros((16,), jnp.float32), v) * iv
        plsc.store_scatter(acc, [idx], v)
        return carry

    lax.fori_loop(0, NPT * d // 16, finstep, 0)
    pltpu.sync_copy(acc, out_hbm.at[pl.ds(nbase * d, NPT * d)])


def _sc_scatter(M_flat, dR, offs, inv, d):
    k = pl.kernel(
        functools.partial(_scatter_body, d=d),
        out_type=jax.ShapeDtypeStruct((N2 * d,), jnp.float32),
        mesh=_MESH(),
        scratch_types=[
            pltpu.VMEM((NPT * d,), jnp.float32),
            pltpu.VMEM((WS,), jnp.int32),
            pltpu.VMEM((WS * d,), jnp.float32),
            pltpu.VMEM((48,), jnp.int32),
            pltpu.VMEM((NPT,), jnp.float32),
            pltpu.SemaphoreType.DMA,
        ],
        compiler_params=pltpu.CompilerParams(use_tc_tiling_on_sc=False, needs_layout_passes=False),
    )
    return k(M_flat, dR, offs, inv)


def kernel(x, geo, t, category, edge_index, W_fourier, sigW, sigb, emb_table,
           enc1W, enc1b, enc2W, enc2b, m1aW, m1ab, m1bW, m1bb,
           m2aW, m2ab, m2bW, m2bb):
    f32 = jnp.float32
    pad_n = lambda a: jnp.pad(a, ((0, N2 - N), (0, 0)))
    xp = pad_n(x)
    geop = pad_n(geo)
    tp = jnp.pad(t, ((0, N2 - N), (0, 0)), constant_values=0.5)
    catp = jnp.pad(category[:, None], ((0, N2 - N), (0, 0)))
    wf = W_fourier[None, :]
    sigb2 = sigb[None, :]
    e1b2 = enc1b[None, :]
    e2b2 = enc2b[None, :]
    W1A = m1aW[:HIDDEN + COND] - m1aW[HIDDEN + COND:]
    W1B = m1aW[HIDDEN + COND:]
    W2A = m2aW[:HIDDEN + COND] - m2aW[HIDDEN + COND:]
    W2B = m2aW[HIDDEN + COND:]

    A1, B1, cond, invstd = _tc_node0(
        xp, geop, tp, catp, wf, sigW, sigb2, emb_table,
        enc1W, e1b2, enc2W, e2b2, W1A, W1B, m1ab[None, :])

    src = edge_index[0].astype(jnp.int32)
    dst = edge_index[1].astype(jnp.int32)
    dst_p = jnp.pad(dst, (0, EPAD_IN - E))
    src_p = jnp.pad(src, (0, EPAD_IN - E))

    counts = _sc_hist(dst_p)
    dR, sR, offs = _sc_route(dst_p, src_p, counts)
    ones = jnp.ones((N2,), f32)

    PA1, PB1 = _sc_gather(A1, B1, dR, sR)
    M1 = _tc_edge(PA1, PB1, m1bW, m1bb[None, :], HIDDEN)
    o1 = _sc_scatter(M1.reshape(-1), dR, offs, ones, HIDDEN)
    o1 = o1.reshape(N2, HIDDEN)

    A2, B2 = _tc_node2(o1, cond, W2A, W2B, m2ab[None, :])

    PA2, PB2 = _sc_gather(A2, B2, dR, sR)
    M2 = _tc_edge(PA2, PB2, m2bW, m2bb[None, :], STATE)
    o2 = _sc_scatter(M2.reshape(-1), dR, offs, invstd.reshape(-1), STATE)
    out = o2.reshape(N2, STATE)
    return out[:N].astype(f32)


# speedup vs baseline: 2.0473x; 1.0635x over previous
import functools

import jax
import jax.numpy as jnp
from jax import lax
from jax.experimental import pallas as pl
from jax.experimental.pallas import tpu as pltpu
from jax.experimental.pallas import tpu_sc as plsc

N = 50000
E = 800000
HIDDEN = 64
EMBED = 16
STATE = 4
SIZE = 2
CLASS_NUM = 10
COND = 2 * EMBED

NW = 32
NPT = 1568
N2 = NW * NPT
EP = 802816
NEG = -3.0e38


def _node0_body(x_ref, geo_ref, t_ref, cat_ref, wf_ref, sigw_ref, sigb_ref,
                emb_ref, e1w_ref, e1b_ref, e2w_ref, e2b_ref,
                w1a_ref, w1b_ref, b1_ref,
                a_ref, b_out_ref, cond_ref, inv_ref):
    ts = t_ref[...]
    xp = ts * wf_ref[...] * (2.0 * jnp.pi)
    four = jnp.concatenate([jnp.sin(xp), jnp.cos(xp)], axis=-1)
    sig = jnp.maximum(jnp.dot(four, sigw_ref[...]) + sigb_ref[...], 0.0)
    cat = cat_ref[...]
    onehot = (cat == lax.broadcasted_iota(jnp.int32, (1, CLASS_NUM), 1)
              ).astype(jnp.float32)
    cls = jnp.maximum(jnp.dot(onehot, emb_ref[...]), 0.0)
    cond = jnp.concatenate([cls, sig], axis=-1)
    h = jnp.concatenate([x_ref[...], geo_ref[...]], axis=-1)
    h = jnp.maximum(jnp.dot(h, e1w_ref[...]) + e1b_ref[...], 0.0)
    h = jnp.maximum(jnp.dot(h, e2w_ref[...]) + e2b_ref[...], 0.0)
    x1 = jnp.concatenate([h, cond], axis=-1)
    a_ref[...] = jnp.dot(x1, w1a_ref[...]) + b1_ref[...]
    b_out_ref[...] = jnp.dot(x1, w1b_ref[...])
    cond_ref[...] = cond
    l25 = jnp.log(25.0)
    std = jnp.sqrt((jnp.exp(2.0 * l25 * ts) - 1.0) / (2.0 * l25))
    inv_ref[...] = 1.0 / (std + 1e-7)


def _tc_node0(xp, geop, tp, catp, wf, sigW, sigb, emb, e1W, e1b, e2W, e2b,
              W1A, W1B, b1):
    BN = 6272
    grid = (N2 // BN,)
    row = lambda i: (i, 0)
    full = lambda i: (0, 0)
    return pl.pallas_call(
        _node0_body,
        grid=grid,
        in_specs=[
            pl.BlockSpec((BN, STATE), row),
            pl.BlockSpec((BN, SIZE), row),
            pl.BlockSpec((BN, 1), row),
            pl.BlockSpec((BN, 1), row),
            pl.BlockSpec((1, EMBED // 2), full),
            pl.BlockSpec((EMBED, EMBED), full),
            pl.BlockSpec((1, EMBED), full),
            pl.BlockSpec((CLASS_NUM, EMBED), full),
            pl.BlockSpec((STATE + SIZE, HIDDEN), full),
            pl.BlockSpec((1, HIDDEN), full),
            pl.BlockSpec((HIDDEN, HIDDEN), full),
            pl.BlockSpec((1, HIDDEN), full),
            pl.BlockSpec((HIDDEN + COND, HIDDEN), full),
            pl.BlockSpec((HIDDEN + COND, HIDDEN), full),
            pl.BlockSpec((1, HIDDEN), full),
        ],
        out_specs=[
            pl.BlockSpec((BN, HIDDEN), row),
            pl.BlockSpec((BN, HIDDEN), row),
            pl.BlockSpec((BN, COND), row),
            pl.BlockSpec((BN, 1), row),
        ],
        out_shape=[
            jax.ShapeDtypeStruct((N2, HIDDEN), jnp.float32),
            jax.ShapeDtypeStruct((N2, HIDDEN), jnp.float32),
            jax.ShapeDtypeStruct((N2, COND), jnp.float32),
            jax.ShapeDtypeStruct((N2, 1), jnp.float32),
        ],
    )(xp, geop, tp, catp, wf, sigW, sigb, emb, e1W, e1b, e2W, e2b, W1A, W1B, b1)


def _node2_body(o1_ref, cond_ref, w2a_ref, w2b_ref, b2_ref, a_ref, b_out_ref):
    x2 = jnp.concatenate([jnp.maximum(o1_ref[...], 0.0), cond_ref[...]],
                         axis=-1)
    a_ref[...] = jnp.dot(x2, w2a_ref[...]) + b2_ref[...]
    b_out_ref[...] = jnp.dot(x2, w2b_ref[...])


def _tc_node2(out1, cond, W2A, W2B, b2):
    BN = 6272
    grid = (N2 // BN,)
    row = lambda i: (i, 0)
    full = lambda i: (0, 0)
    return pl.pallas_call(
        _node2_body,
        grid=grid,
        in_specs=[
            pl.BlockSpec((BN, HIDDEN), row),
            pl.BlockSpec((BN, COND), row),
            pl.BlockSpec((HIDDEN + COND, HIDDEN), full),
            pl.BlockSpec((HIDDEN + COND, HIDDEN), full),
            pl.BlockSpec((1, HIDDEN), full),
        ],
        out_specs=[
            pl.BlockSpec((BN, HIDDEN), row),
            pl.BlockSpec((BN, HIDDEN), row),
        ],
        out_shape=[
            jax.ShapeDtypeStruct((N2, HIDDEN), jnp.float32),
            jax.ShapeDtypeStruct((N2, HIDDEN), jnp.float32),
        ],
    )(out1, cond, W2A, W2B, b2)


def _edge_body(pa_ref, pb_ref, w_ref, b_ref, m_ref):
    g = jnp.maximum(pa_ref[...] + pb_ref[...], 0.0)
    m_ref[...] = jnp.dot(g, w_ref[...]) + b_ref[...]


def _tc_edge(PA, PB, W2, b2, dout):
    BE = 2048
    grid = (EP // BE,)
    row = lambda i: (i, 0)
    full = lambda i: (0, 0)
    return pl.pallas_call(
        _edge_body,
        grid=grid,
        in_specs=[
            pl.BlockSpec((BE, HIDDEN), row),
            pl.BlockSpec((BE, HIDDEN), row),
            pl.BlockSpec((HIDDEN, dout), full),
            pl.BlockSpec((1, dout), full),
        ],
        out_specs=pl.BlockSpec((BE, dout), row),
        out_shape=jax.ShapeDtypeStruct((EP, dout), jnp.float32),
    )(PA, PB, W2, b2)


_MESH = functools.partial(plsc.VectorSubcoreMesh, core_axis_name="c",
                          subcore_axis_name="s", num_cores=2, num_subcores=16)

EW = E // NW
GW = 256
GEW = 25088
NWG = 98


def _sc_gather_body(a_hbm, b_hbm, d_hbm, s_hbm, pa_hbm, pb_hbm,
                    idxd, idxs, rowsa, rowsb, semid, semis, sema, semb,
                    semoa, semob):
    wid = lax.axis_index("s") * 2 + lax.axis_index("c")
    base = wid * GEW
    nw = jnp.where(wid == NW - 1, 87, NWG)

    def idx_start(k, b):
        s = base + k * GW
        pltpu.async_copy(d_hbm.at[pl.ds(s, GW)], idxd[b], semid[b])
        pltpu.async_copy(s_hbm.at[pl.ds(s, GW)], idxs[b], semis[b])

    def idx_wait(b):
        pltpu.make_async_copy(d_hbm.at[pl.ds(0, GW)], idxd[b], semid[b]).wait()
        pltpu.make_async_copy(s_hbm.at[pl.ds(0, GW)], idxs[b], semis[b]).wait()

    def g_start(b):
        pltpu.async_copy(a_hbm.at[idxd[b]], rowsa[b], sema[b])
        pltpu.async_copy(b_hbm.at[idxs[b]], rowsb[b], semb[b])

    def g_wait(b):
        pltpu.make_async_copy(a_hbm.at[idxd[b]], rowsa[b], sema[b]).wait()
        pltpu.make_async_copy(b_hbm.at[idxs[b]], rowsb[b], semb[b]).wait()

    def out_start(k, b):
        s = base + k * GW
        pltpu.async_copy(rowsa[b], pa_hbm.at[pl.ds(s, GW), :], semoa[b])
        pltpu.async_copy(rowsb[b], pb_hbm.at[pl.ds(s, GW), :], semob[b])

    def out_wait(b):
        pltpu.make_async_copy(rowsa[b], pa_hbm.at[pl.ds(0, GW), :],
                              semoa[b]).wait()
        pltpu.make_async_copy(rowsb[b], pb_hbm.at[pl.ds(0, GW), :],
                              semob[b]).wait()

    idx_start(0, 0)

    def pair(kp, carry):
        for bb in range(2):
            k = kp * 2 + bb

            @pl.when(k < nw)
            def _():
                @pl.when(k + 1 < nw)
                def _():
                    idx_start(k + 1, 1 - bb)

                idx_wait(bb)

                @pl.when(k >= 2)
                def _():
                    out_wait(bb)

                g_start(bb)
                g_wait(bb)
                out_start(k, bb)

        return carry

    lax.fori_loop(0, NWG // 2, pair, 0)

    out_wait(0)
    out_wait(1)


def _sc_gather(A, B, dR, sR):
    k = pl.kernel(
        _sc_gather_body,
        out_type=[
            jax.ShapeDtypeStruct((EP, HIDDEN), jnp.float32),
            jax.ShapeDtypeStruct((EP, HIDDEN), jnp.float32),
        ],
        mesh=_MESH(),
        scratch_types=[
            [pltpu.VMEM((GW,), jnp.int32)] * 2,
            [pltpu.VMEM((GW,), jnp.int32)] * 2,
            [pltpu.VMEM((GW, HIDDEN), jnp.float32)] * 2,
            [pltpu.VMEM((GW, HIDDEN), jnp.float32)] * 2,
            [pltpu.SemaphoreType.DMA] * 2,
            [pltpu.SemaphoreType.DMA] * 2,
            [pltpu.SemaphoreType.DMA] * 2,
            [pltpu.SemaphoreType.DMA] * 2,
            [pltpu.SemaphoreType.DMA] * 2,
            [pltpu.SemaphoreType.DMA] * 2,
        ],
        compiler_params=pltpu.CompilerParams(use_tc_tiling_on_sc=False, needs_layout_passes=False),
    )
    return k(A, B, dR, sR)


HW = 1024
NHG = HW // 16
NWIN = (EW + HW - 1) // HW
EPAD_IN = E + HW * NW // 32 + 1024


def _bucket(d):
    return ((d >> 5) * 2675) >> 17


def _iota():
    return lax.iota(jnp.int32, 16)


def _sc_hist_body(d_hbm, cnt_hbm, loc, win, sem):
    wid = lax.axis_index("s") * 2 + lax.axis_index("c")
    base = wid * EW
    zero = jnp.zeros((16,), jnp.int32)
    for b in range(32):
        loc[pl.ds(b * 16, 16)] = zero

    def step(g, carry):
        pltpu.sync_copy(d_hbm.at[pl.ds(base + g * HW, HW)], win)
        rem = EW - g * HW

        def group(i, c2):
            msk = (i * 16 + _iota()) < rem
            d = win[pl.ds(i * 16, 16)]
            addr = _bucket(d) * 16 + _iota()
            cur = plsc.load_gather(loc, [addr], mask=msk)
            plsc.store_scatter(loc, [addr], cur + 1, mask=msk)
            return c2

        lax.fori_loop(0, NHG, group, 0)
        return carry

    lax.fori_loop(0, NWIN, step, 0)
    for b in range(32):
        pltpu.sync_copy(loc.at[pl.ds(b * 16, 16)],
                        cnt_hbm.at[pl.ds(b * NW * 16 + wid * 16, 16)])


def _sc_hist(dst_p):
    k = pl.kernel(
        _sc_hist_body,
        out_type=jax.ShapeDtypeStruct((32 * NW * 16,), jnp.int32),
        mesh=_MESH(),
        scratch_types=[
            pltpu.VMEM((32 * 16,), jnp.int32),
            pltpu.VMEM((HW,), jnp.int32),
            pltpu.SemaphoreType.DMA,
        ],
        compiler_params=pltpu.CompilerParams(use_tc_tiling_on_sc=False, needs_layout_passes=False),
    )
    return k(dst_p)


def _sc_route_body(d_hbm, s_hbm, cnt_hbm, dr_hbm, sr_hbm, offs_hbm,
                   scan, dwin, swin, pos, offsv, sem, semd, sems):
    wid = lax.axis_index("s") * 2 + lax.axis_index("c")
    base = wid * EW
    pltpu.sync_copy(cnt_hbm, scan)

    def scan_step(k, carry):
        v = plsc.load_gather(scan, [k * 16 + _iota()])
        s = plsc.cumsum(v)
        excl = s - v + carry
        plsc.store_scatter(scan, [k * 16 + _iota()], excl)
        return carry + lax.reduce_max(s, (0,))

    lax.fori_loop(0, 32 * NW, scan_step, jnp.int32(0))

    o0 = plsc.load_gather(scan, [_iota() * (NW * 16)])
    o1 = plsc.load_gather(scan, [(16 + _iota()) * (NW * 16)])
    offsv[pl.ds(0, 16)] = o0
    offsv[pl.ds(16, 16)] = o1
    offsv[pl.ds(32, 16)] = jnp.full((16,), E, jnp.int32)

    @pl.when(wid == 0)
    def _():
        pltpu.sync_copy(offsv, offs_hbm)

    trash = E + wid * 16 + _iota()

    def step(g, carry):
        gbase = base + g * HW
        pltpu.sync_copy(d_hbm.at[pl.ds(gbase, HW)], dwin)
        pltpu.sync_copy(s_hbm.at[pl.ds(gbase, HW)], swin)
        rem = EW - g * HW

        def group(i, c2):
            msk = (i * 16 + _iota()) < rem
            d = dwin[pl.ds(i * 16, 16)]
            addr = _bucket(d) * (NW * 16) + wid * 16 + _iota()
            p = plsc.load_gather(scan, [addr], mask=msk)
            plsc.store_scatter(scan, [addr], p + 1, mask=msk)
            plsc.store_scatter(pos, [i * 16 + _iota()], jnp.where(msk, p, trash))
            return c2

        lax.fori_loop(0, NHG, group, 0)
        cpd = pltpu.async_copy(dwin, dr_hbm.at[pos], semd)
        cps = pltpu.async_copy(swin, sr_hbm.at[pos], sems)
        cpd.wait()
        cps.wait()
        return carry

    lax.fori_loop(0, NWIN, step, 0)


def _sc_route(dst_p, src_p, counts):
    k = pl.kernel(
        _sc_route_body,
        out_type=[
            jax.ShapeDtypeStruct((EP,), jnp.int32),
            jax.ShapeDtypeStruct((EP,), jnp.int32),
            jax.ShapeDtypeStruct((48,), jnp.int32),
        ],
        mesh=_MESH(),
        scratch_types=[
            pltpu.VMEM((32 * NW * 16,), jnp.int32),
            pltpu.VMEM((HW,), jnp.int32),
            pltpu.VMEM((HW,), jnp.int32),
            pltpu.VMEM((HW,), jnp.int32),
            pltpu.VMEM((48,), jnp.int32),
            pltpu.SemaphoreType.DMA,
            pltpu.SemaphoreType.DMA,
            pltpu.SemaphoreType.DMA,
        ],
        compiler_params=pltpu.CompilerParams(use_tc_tiling_on_sc=False, needs_layout_passes=False),
    )
    return k(dst_p, src_p, counts)


WS = 192
NSG = WS // 16


def _scatter_body(m_hbm, dr_hbm, offs_hbm, inv_hbm, out_hbm,
                  acc, dwin, mwin, offsv, invv, semd, semm, d):
    wid = lax.axis_index("s") * 2 + lax.axis_index("c")
    nbase = wid * NPT
    pltpu.sync_copy(offs_hbm, offsv)
    pltpu.sync_copy(inv_hbm.at[pl.ds(nbase, NPT)], invv)
    widv = jnp.full((16,), wid, jnp.int32)
    off_s = lax.reduce_max(plsc.load_gather(offsv, [widv]), (0,))
    end_s = lax.reduce_max(plsc.load_gather(offsv, [widv + 1]), (0,))
    start0 = (off_s // 8) * 8
    nwin = (end_s - start0 + WS - 1) // WS

    ninf = jnp.full((16,), -jnp.inf, jnp.float32)

    def initstep(k, carry):
        plsc.store_scatter(acc, [k * 16 + _iota()], ninf)
        return carry

    lax.fori_loop(0, NPT * d // 16, initstep, 0)

    def w_start(g, b):
        wbase = start0 + g * WS
        pltpu.async_copy(dr_hbm.at[pl.ds(wbase, WS)], dwin[b], semd[b])
        pltpu.async_copy(m_hbm.at[pl.ds(wbase * d, WS * d)], mwin[b], semm[b])

    def w_wait(b):
        pltpu.make_async_copy(dr_hbm.at[pl.ds(0, WS)], dwin[b],
                              semd[b]).wait()
        pltpu.make_async_copy(m_hbm.at[pl.ds(0, WS * d)], mwin[b],
                              semm[b]).wait()

    @pl.when(nwin > 0)
    def _():
        w_start(0, 0)

    def rmw(g, b):
        wbase = start0 + g * WS

        def group(i, c2):
            ebase = wbase + i * 16
            for j in range(16):
                ok = jnp.logical_and(ebase + j >= off_s, ebase + j < end_s)
                mskj = jnp.full((16,), False) | ok
                ev = jnp.full((16,), i * 16 + j, jnp.int32)
                dj = plsc.load_gather(dwin[b], [ev]) - nbase
                rowm = (i * 16 + j) * d
                if d >= 16:
                    for c in range(d // 16):
                        addr = dj * d + c * 16 + _iota()
                        cur = plsc.load_gather(acc, [addr], mask=mskj)
                        mv = mwin[b][pl.ds(rowm + c * 16, 16)]
                        plsc.store_scatter(acc, [addr],
                                           jnp.maximum(cur, mv), mask=mskj)
                else:
                    mk = jnp.logical_and(mskj, _iota() < d)
                    addr = dj * d + _iota()
                    cur = plsc.load_gather(acc, [addr], mask=mk)
                    mv = plsc.load_gather(mwin[b], [rowm + _iota()], mask=mk)
                    plsc.store_scatter(acc, [addr],
                                       jnp.maximum(cur, mv), mask=mk)
            return c2

        lax.fori_loop(0, NSG, group, 0)

    def pair(gp, carry):
        for bb in range(2):
            g = gp * 2 + bb

            @pl.when(g < nwin)
            def _():
                w_wait(bb)

                @pl.when(g + 1 < nwin)
                def _():
                    w_start(g + 1, 1 - bb)

                rmw(g, bb)

        return carry

    lax.fori_loop(0, nwin // 2 + 1, pair, 0)

    def finstep(k, carry):
        idx = k * 16 + _iota()
        v = plsc.load_gather(acc, [idx])
        iv = plsc.load_gather(invv, [idx // d])
        v = jnp.where(v == ninf, jnp.zeros((16,), jnp.float32), v) * iv
        plsc.store_scatter(acc, [idx], v)
        return carry

    lax.fori_loop(0, NPT * d // 16, finstep, 0)
    pltpu.sync_copy(acc, out_hbm.at[pl.ds(nbase * d, NPT * d)])


def _sc_scatter(M_flat, dR, offs, inv, d):
    k = pl.kernel(
        functools.partial(_scatter_body, d=d),
        out_type=jax.ShapeDtypeStruct((N2 * d,), jnp.float32),
        mesh=_MESH(),
        scratch_types=[
            pltpu.VMEM((NPT * d,), jnp.float32),
            [pltpu.VMEM((WS,), jnp.int32)] * 2,
            [pltpu.VMEM((WS * d,), jnp.float32)] * 2,
            pltpu.VMEM((48,), jnp.int32),
            pltpu.VMEM((NPT,), jnp.float32),
            [pltpu.SemaphoreType.DMA] * 2,
            [pltpu.SemaphoreType.DMA] * 2,
        ],
        compiler_params=pltpu.CompilerParams(use_tc_tiling_on_sc=False, needs_layout_passes=False),
    )
    return k(M_flat, dR, offs, inv)


def kernel(x, geo, t, category, edge_index, W_fourier, sigW, sigb, emb_table,
           enc1W, enc1b, enc2W, enc2b, m1aW, m1ab, m1bW, m1bb,
           m2aW, m2ab, m2bW, m2bb):
    f32 = jnp.float32
    pad_n = lambda a: jnp.pad(a, ((0, N2 - N), (0, 0)))
    xp = pad_n(x)
    geop = pad_n(geo)
    tp = jnp.pad(t, ((0, N2 - N), (0, 0)), constant_values=0.5)
    catp = jnp.pad(category[:, None], ((0, N2 - N), (0, 0)))
    wf = W_fourier[None, :]
    sigb2 = sigb[None, :]
    e1b2 = enc1b[None, :]
    e2b2 = enc2b[None, :]
    W1A = m1aW[:HIDDEN + COND] - m1aW[HIDDEN + COND:]
    W1B = m1aW[HIDDEN + COND:]
    W2A = m2aW[:HIDDEN + COND] - m2aW[HIDDEN + COND:]
    W2B = m2aW[HIDDEN + COND:]

    A1, B1, cond, invstd = _tc_node0(
        xp, geop, tp, catp, wf, sigW, sigb2, emb_table,
        enc1W, e1b2, enc2W, e2b2, W1A, W1B, m1ab[None, :])

    src = edge_index[0].astype(jnp.int32)
    dst = edge_index[1].astype(jnp.int32)
    dst_p = jnp.pad(dst, (0, EPAD_IN - E))
    src_p = jnp.pad(src, (0, EPAD_IN - E))

    counts = _sc_hist(dst_p)
    dR, sR, offs = _sc_route(dst_p, src_p, counts)
    ones = jnp.ones((N2,), f32)

    PA1, PB1 = _sc_gather(A1, B1, dR, sR)
    M1 = _tc_edge(PA1, PB1, m1bW, m1bb[None, :], HIDDEN)
    o1 = _sc_scatter(M1.reshape(-1), dR, offs, ones, HIDDEN)
    o1 = o1.reshape(N2, HIDDEN)

    A2, B2 = _tc_node2(o1, cond, W2A, W2B, m2ab[None, :])

    PA2, PB2 = _sc_gather(A2, B2, dR, sR)
    M2 = _tc_edge(PA2, PB2, m2bW, m2bb[None, :], STATE)
    o2 = _sc_scatter(M2.reshape(-1), dR, offs, invstd.reshape(-1), STATE)
    out = o2.reshape(N2, STATE)
    return out[:N].astype(f32)
